# Initial kernel scaffold; baseline (speedup 1.0000x reference)
import jax
import jax.numpy as jnp
from jax.experimental import pallas as pl


def kernel(seq, edge_index, edge_weight, sparse, msk, W1, b1, a1, W2, b2, a2):
    raise NotImplementedError("write your pallas kernel here")


# speedup vs baseline: 6.0477x; 6.0477x over previous
import functools

import jax
import jax.numpy as jnp
from jax import lax
from jax.experimental import pallas as pl
from jax.experimental.pallas import tpu as pltpu
from jax.experimental.pallas import tpu_sc as plsc


def _spmm_call(x, src, dst, w, zeros):
    n, f = x.shape
    e = src.shape[0]
    nw = 32
    epw = e // nw
    blk = 80
    nch = epw // blk
    rpt8 = (n // 16) // 8 * 8
    last = n - 15 * rpt8
    mesh = plsc.VectorSubcoreMesh(core_axis_name="c", subcore_axis_name="s")

    @functools.partial(
        pl.kernel,
        mesh=mesh,
        out_type=jax.ShapeDtypeStruct((2, n, f), jnp.float32),
        scratch_types=[
            pltpu.VMEM((blk,), jnp.int32),
            pltpu.VMEM((blk,), jnp.int32),
            pltpu.VMEM((blk,), jnp.float32),
            pltpu.VMEM((blk, f), jnp.float32),
            pltpu.VMEM_SHARED((n, f), jnp.float32),
            pltpu.SemaphoreType.DMA,
        ],
    )
    def k(x_hbm, src_hbm, dst_hbm, w_hbm, z_hbm, out_hbm,
          sidx, didx, wv, rows, acc, sem):
        cid = lax.axis_index("c")
        sid = lax.axis_index("s")
        wid = sid * 2 + cid
        @pl.when(sid < 15)
        def _():
            pltpu.sync_copy(z_hbm.at[pl.ds(sid * rpt8, rpt8)],
                            acc.at[pl.ds(sid * rpt8, rpt8)])

        @pl.when(sid == 15)
        def _():
            pltpu.sync_copy(z_hbm.at[pl.ds(15 * rpt8, last)],
                            acc.at[pl.ds(15 * rpt8, last)])

        plsc.subcore_barrier()
        ebase = wid * epw

        def chunk(ci, carry):
            off = ebase + ci * blk
            pltpu.sync_copy(src_hbm.at[pl.ds(off, blk)], sidx)
            pltpu.sync_copy(dst_hbm.at[pl.ds(off, blk)], didx)
            pltpu.sync_copy(w_hbm.at[pl.ds(off, blk)], wv)
            pltpu.async_copy(x_hbm.at[sidx], rows, sem).wait()

            def grp(g, c2):
                wgrp = wv[pl.ds(g * 16, 16)]
                for r2 in range(16):
                    wvec = jnp.full((16,), wgrp[r2], jnp.float32)
                    r = g * 16 + r2
                    for j in range(f // 16):
                        sl = pl.ds(j * 16, 16)
                        rows[r, sl] = rows[r, sl] * wvec
                return c2

            lax.fori_loop(0, blk // 16, grp, 0)
            pltpu.sync_copy(rows, acc.at[didx], add=True)
            return carry

        lax.fori_loop(0, nch, chunk, 0)
        plsc.subcore_barrier()

        @pl.when(sid < 15)
        def _():
            pltpu.sync_copy(acc.at[pl.ds(sid * rpt8, rpt8)],
                            out_hbm.at[cid, pl.ds(sid * rpt8, rpt8)])

        @pl.when(sid == 15)
        def _():
            pltpu.sync_copy(acc.at[pl.ds(15 * rpt8, last)],
                            out_hbm.at[cid, pl.ds(15 * rpt8, last)])

    return k(x, src, dst, w, zeros)


def _mid_call(parts, W1, b1, a1, W2):
    _, n, f = parts.shape
    h = W1.shape[1]
    bm = 400
    gm = n // bm

    def body(p_ref, w1_ref, b1_ref, w2_ref, a1_ref, y_ref):
        agg = p_ref[0] + p_ref[1]
        t = jnp.dot(agg, w1_ref[...], preferred_element_type=jnp.float32)
        t = t + b1_ref[...]
        al = a1_ref[0]
        t = jnp.where(t >= 0, t, al * t)
        y_ref[...] = jnp.dot(t, w2_ref[...], preferred_element_type=jnp.float32)

    return pl.pallas_call(
        body,
        grid=(gm,),
        in_specs=[
            pl.BlockSpec((2, bm, f), lambda i: (0, i, 0)),
            pl.BlockSpec((f, h), lambda i: (0, 0)),
            pl.BlockSpec((1, h), lambda i: (0, 0)),
            pl.BlockSpec((h, f), lambda i: (0, 0)),
            pl.BlockSpec(memory_space=pltpu.SMEM),
        ],
        out_specs=pl.BlockSpec((bm, f), lambda i: (i, 0)),
        out_shape=jax.ShapeDtypeStruct((n, f), jnp.float32),
    )(parts, W1, b1.reshape(1, h), W2, a1.reshape(1))


def _final_call(parts, b2, a2, mskc):
    _, n, f = parts.shape
    bm = 400
    gm = n // bm

    def body(p_ref, b2_ref, m_ref, a2_ref, h_ref, cs_ref, ms_ref):
        i = pl.program_id(0)
        t = p_ref[0] + p_ref[1] + b2_ref[...]
        al = a2_ref[0]
        hh = jnp.where(t >= 0, t, al * t)
        h_ref[...] = hh
        m = m_ref[...]

        @pl.when(i == 0)
        def _():
            cs_ref[...] = jnp.zeros_like(cs_ref)
            ms_ref[...] = jnp.zeros_like(ms_ref)

        cs_ref[...] += jnp.sum(hh * m, axis=0, keepdims=True)
        ms_ref[...] += jnp.sum(m, axis=0, keepdims=True)

    return pl.pallas_call(
        body,
        grid=(gm,),
        in_specs=[
            pl.BlockSpec((2, bm, f), lambda i: (0, i, 0)),
            pl.BlockSpec((1, f), lambda i: (0, 0)),
            pl.BlockSpec((bm, 1), lambda i: (i, 0)),
            pl.BlockSpec(memory_space=pltpu.SMEM),
        ],
        out_specs=[
            pl.BlockSpec((bm, f), lambda i: (i, 0)),
            pl.BlockSpec((1, f), lambda i: (0, 0)),
            pl.BlockSpec((1, 1), lambda i: (0, 0)),
        ],
        out_shape=[
            jax.ShapeDtypeStruct((n, f), jnp.float32),
            jax.ShapeDtypeStruct((1, f), jnp.float32),
            jax.ShapeDtypeStruct((1, 1), jnp.float32),
        ],
    )(parts, b2.reshape(1, f), mskc, a2.reshape(1))


def kernel(seq, edge_index, edge_weight, sparse, msk, W1, b1, a1, W2, b2, a2):
    n, f = seq.shape[1], seq.shape[2]
    x0 = seq[0]
    src = edge_index[1]
    dst = edge_index[0]
    zeros = jnp.zeros((n, f), jnp.float32)
    p1 = _spmm_call(x0, src, dst, edge_weight, zeros)
    y = _mid_call(p1, W1, b1, a1, W2)
    p2 = _spmm_call(y, src, dst, edge_weight, zeros)
    hout, cs, ms = _final_call(p2, b2, a2, msk.reshape(n, 1))
    return (hout[None], cs / ms)


# speedup vs baseline: 13.8884x; 2.2965x over previous
import functools

import jax
import jax.numpy as jnp
from jax import lax
from jax.experimental import pallas as pl
from jax.experimental.pallas import tpu as pltpu
from jax.experimental.pallas import tpu_sc as plsc


def _spmm_call(x, src, dst, w, zeros):
    n, f = x.shape
    e = src.shape[0]
    nw = 32
    epw = e // nw
    blk = 80
    nch = epw // blk
    ngrp = 5
    grp = nch // ngrp
    rpt8 = (n // 16) // 8 * 8
    last = n - 15 * rpt8
    mesh = plsc.VectorSubcoreMesh(core_axis_name="c", subcore_axis_name="s")

    @functools.partial(
        pl.kernel,
        mesh=mesh,
        out_type=jax.ShapeDtypeStruct((2, n, f), jnp.float32),
        scratch_types=[
            pltpu.VMEM((1, 1, grp, blk), jnp.int32),
            pltpu.VMEM((1, 1, grp, blk), jnp.int32),
            pltpu.VMEM((1, 1, grp, blk), jnp.float32),
            pltpu.VMEM((blk, f), jnp.float32),
            pltpu.VMEM((blk, f), jnp.float32),
            pltpu.VMEM_SHARED((n, f), jnp.float32),
            pltpu.SemaphoreType.DMA,
            pltpu.SemaphoreType.DMA,
        ],
    )
    def k(x_hbm, src_hbm, dst_hbm, w_hbm, z_hbm, out_hbm,
          sidx, didx, wv, rows0, rows1, acc, gs0, gs1):
        cid = lax.axis_index("c")
        sid = lax.axis_index("s")
        wid = sid * 2 + cid
        @pl.when(sid < 15)
        def _():
            pltpu.sync_copy(z_hbm.at[pl.ds(sid * rpt8, rpt8)],
                            acc.at[pl.ds(sid * rpt8, rpt8)])

        @pl.when(sid == 15)
        def _():
            pltpu.sync_copy(z_hbm.at[pl.ds(15 * rpt8, last)],
                            acc.at[pl.ds(15 * rpt8, last)])

        plsc.subcore_barrier()

        def scale(rows, ci):
            def sgrp(g, c2):
                wgrp = wv[0, 0, ci, pl.ds(g * 16, 16)]
                for r2 in range(16):
                    wvec = jnp.full((16,), wgrp[r2], jnp.float32)
                    r = g * 16 + r2
                    for j in range(f // 16):
                        sl = pl.ds(j * 16, 16)
                        rows[r, sl] = rows[r, sl] * wvec
                return c2

            lax.fori_loop(0, blk // 16, sgrp, 0)

        def group(g, carry):
            pltpu.sync_copy(src_hbm.at[pl.ds(wid, 1), pl.ds(g, 1)], sidx)
            pltpu.sync_copy(dst_hbm.at[pl.ds(wid, 1), pl.ds(g, 1)], didx)
            pltpu.sync_copy(w_hbm.at[pl.ds(wid, 1), pl.ds(g, 1)], wv)
            pltpu.async_copy(x_hbm.at[sidx.at[0, 0, 0]], rows0, gs0)
            pltpu.async_copy(x_hbm.at[sidx.at[0, 0, 1]], rows1, gs1)

            def stage(ci, rows, gs):
                pltpu.make_async_copy(x_hbm.at[sidx.at[0, 0, ci]], rows, gs).wait()
                scale(rows, ci)
                pltpu.sync_copy(rows, acc.at[didx.at[0, 0, ci]], add=True)

                @pl.when(ci + 2 < grp)
                def _():
                    pltpu.async_copy(x_hbm.at[sidx.at[0, 0, ci + 2]], rows, gs)

            def pair(p, c2):
                c0 = 2 * p
                stage(c0, rows0, gs0)

                @pl.when(c0 + 1 < grp)
                def _():
                    stage(c0 + 1, rows1, gs1)

                return c2

            lax.fori_loop(0, (grp + 1) // 2, pair, 0)
            return carry

        lax.fori_loop(0, ngrp, group, 0)
        plsc.subcore_barrier()

        @pl.when(sid < 15)
        def _():
            pltpu.sync_copy(acc.at[pl.ds(sid * rpt8, rpt8)],
                            out_hbm.at[cid, pl.ds(sid * rpt8, rpt8)])

        @pl.when(sid == 15)
        def _():
            pltpu.sync_copy(acc.at[pl.ds(15 * rpt8, last)],
                            out_hbm.at[cid, pl.ds(15 * rpt8, last)])

    return k(x, src.reshape(nw, ngrp, grp, blk), dst.reshape(nw, ngrp, grp, blk),
             w.reshape(nw, ngrp, grp, blk), zeros)


def _mid_call(parts, W1, b1, a1, W2):
    _, n, f = parts.shape
    h = W1.shape[1]
    bm = 400
    gm = n // bm

    def body(p_ref, w1_ref, b1_ref, w2_ref, a1_ref, y_ref):
        agg = p_ref[0] + p_ref[1]
        t = jnp.dot(agg, w1_ref[...], preferred_element_type=jnp.float32)
        t = t + b1_ref[...]
        al = a1_ref[0]
        t = jnp.where(t >= 0, t, al * t)
        y_ref[...] = jnp.dot(t, w2_ref[...], preferred_element_type=jnp.float32)

    return pl.pallas_call(
        body,
        grid=(gm,),
        in_specs=[
            pl.BlockSpec((2, bm, f), lambda i: (0, i, 0)),
            pl.BlockSpec((f, h), lambda i: (0, 0)),
            pl.BlockSpec((1, h), lambda i: (0, 0)),
            pl.BlockSpec((h, f), lambda i: (0, 0)),
            pl.BlockSpec(memory_space=pltpu.SMEM),
        ],
        out_specs=pl.BlockSpec((bm, f), lambda i: (i, 0)),
        out_shape=jax.ShapeDtypeStruct((n, f), jnp.float32),
    )(parts, W1, b1.reshape(1, h), W2, a1.reshape(1))


def _final_call(parts, b2, a2, mskc):
    _, n, f = parts.shape
    bm = 400
    gm = n // bm

    def body(p_ref, b2_ref, m_ref, a2_ref, h_ref, cs_ref, ms_ref):
        i = pl.program_id(0)
        t = p_ref[0] + p_ref[1] + b2_ref[...]
        al = a2_ref[0]
        hh = jnp.where(t >= 0, t, al * t)
        h_ref[...] = hh
        m = m_ref[...]

        @pl.when(i == 0)
        def _():
            cs_ref[...] = jnp.zeros_like(cs_ref)
            ms_ref[...] = jnp.zeros_like(ms_ref)

        cs_ref[...] += jnp.sum(hh * m, axis=0, keepdims=True)
        ms_ref[...] += jnp.sum(m, axis=0, keepdims=True)

    return pl.pallas_call(
        body,
        grid=(gm,),
        in_specs=[
            pl.BlockSpec((2, bm, f), lambda i: (0, i, 0)),
            pl.BlockSpec((1, f), lambda i: (0, 0)),
            pl.BlockSpec((bm, 1), lambda i: (i, 0)),
            pl.BlockSpec(memory_space=pltpu.SMEM),
        ],
        out_specs=[
            pl.BlockSpec((bm, f), lambda i: (i, 0)),
            pl.BlockSpec((1, f), lambda i: (0, 0)),
            pl.BlockSpec((1, 1), lambda i: (0, 0)),
        ],
        out_shape=[
            jax.ShapeDtypeStruct((n, f), jnp.float32),
            jax.ShapeDtypeStruct((1, f), jnp.float32),
            jax.ShapeDtypeStruct((1, 1), jnp.float32),
        ],
    )(parts, b2.reshape(1, f), mskc, a2.reshape(1))


def kernel(seq, edge_index, edge_weight, sparse, msk, W1, b1, a1, W2, b2, a2):
    n, f = seq.shape[1], seq.shape[2]
    x0 = seq[0]
    src = edge_index[1]
    dst = edge_index[0]
    zeros = jnp.zeros((n, f), jnp.float32)
    p1 = _spmm_call(x0, src, dst, edge_weight, zeros)
    y = _mid_call(p1, W1, b1, a1, W2)
    p2 = _spmm_call(y, src, dst, edge_weight, zeros)
    hout, cs, ms = _final_call(p2, b2, a2, msk.reshape(n, 1))
    return (hout[None], cs / ms)


# speedup vs baseline: 15.3371x; 1.1043x over previous
import functools

import jax
import jax.numpy as jnp
from jax import lax
from jax.experimental import pallas as pl
from jax.experimental.pallas import tpu as pltpu
from jax.experimental.pallas import tpu_sc as plsc


def _spmm_call(x, src, dst, w, zeros):
    n, f = x.shape
    e = src.shape[0]
    nw = 32
    epw = e // nw
    blk = 80
    nch = epw // blk
    ngrp = 5
    grp = nch // ngrp
    rpt8 = (n // 16) // 8 * 8
    last = n - 15 * rpt8
    mesh = plsc.VectorSubcoreMesh(core_axis_name="c", subcore_axis_name="s")

    @functools.partial(
        pl.kernel,
        mesh=mesh,
        out_type=jax.ShapeDtypeStruct((2, n, f), jnp.float32),
        scratch_types=[
            pltpu.VMEM((1, 1, grp, blk), jnp.int32),
            pltpu.VMEM((1, 1, grp, blk), jnp.int32),
            pltpu.VMEM((1, 1, grp, blk), jnp.float32),
            pltpu.VMEM((blk, f), jnp.float32),
            pltpu.VMEM((blk, f), jnp.float32),
            pltpu.VMEM((blk, f), jnp.float32),
            pltpu.VMEM_SHARED((n, f), jnp.float32),
            pltpu.SemaphoreType.DMA,
            pltpu.SemaphoreType.DMA,
            pltpu.SemaphoreType.DMA,
            pltpu.SemaphoreType.DMA,
            pltpu.SemaphoreType.DMA,
            pltpu.SemaphoreType.DMA,
        ],
    )
    def k(x_hbm, src_hbm, dst_hbm, w_hbm, z_hbm, out_hbm,
          sidx, didx, wv, rows0, rows1, rows2, acc,
          gs0, gs1, gs2, ss0, ss1, ss2):
        cid = lax.axis_index("c")
        sid = lax.axis_index("s")
        wid = sid * 2 + cid
        @pl.when(sid < 15)
        def _():
            pltpu.sync_copy(z_hbm.at[pl.ds(sid * rpt8, rpt8)],
                            acc.at[pl.ds(sid * rpt8, rpt8)])

        @pl.when(sid == 15)
        def _():
            pltpu.sync_copy(z_hbm.at[pl.ds(15 * rpt8, last)],
                            acc.at[pl.ds(15 * rpt8, last)])

        plsc.subcore_barrier()

        def scale(rows, ci):
            def sgrp(g, c2):
                wgrp = wv[0, 0, ci, pl.ds(g * 16, 16)]
                for r2 in range(16):
                    wvec = jnp.full((16,), wgrp[r2], jnp.float32)
                    r = g * 16 + r2
                    for j in range(f // 16):
                        sl = pl.ds(j * 16, 16)
                        rows[r, sl] = rows[r, sl] * wvec
                return c2

            lax.fori_loop(0, blk // 16, sgrp, 0)

        def group(g, carry):
            pltpu.sync_copy(src_hbm.at[pl.ds(wid, 1), pl.ds(g, 1)], sidx)
            pltpu.sync_copy(dst_hbm.at[pl.ds(wid, 1), pl.ds(g, 1)], didx)
            pltpu.sync_copy(w_hbm.at[pl.ds(wid, 1), pl.ds(g, 1)], wv)
            rbufs = (rows0, rows1, rows2)
            gsems = (gs0, gs1, gs2)
            ssems = (ss0, ss1, ss2)
            pltpu.async_copy(x_hbm.at[sidx.at[0, 0, 0]], rows0, gs0)
            pltpu.async_copy(x_hbm.at[sidx.at[0, 0, 1]], rows1, gs1)

            def stage(ci, b):
                rows, gs, ss = rbufs[b], gsems[b], ssems[b]
                nb = (b + 2) % 3
                pltpu.make_async_copy(x_hbm.at[sidx.at[0, 0, ci]], rows, gs).wait()
                scale(rows, ci)
                pltpu.async_copy(rows, acc.at[didx.at[0, 0, ci]], ss, add=True)

                @pl.when(ci + 2 < grp)
                def _():
                    @pl.when(ci >= 1)
                    def _():
                        cim1 = jnp.maximum(ci - 1, 0)
                        pltpu.make_async_copy(
                            rbufs[nb], acc.at[didx.at[0, 0, cim1]],
                            ssems[nb]).wait()

                    pltpu.async_copy(x_hbm.at[sidx.at[0, 0, ci + 2]],
                                     rbufs[nb], gsems[nb])

            def triple(t, c2):
                for k3 in range(3):
                    ci = 3 * t + k3

                    @pl.when(ci < grp)
                    def _():
                        stage(ci, k3)

                return c2

            lax.fori_loop(0, (grp + 2) // 3, triple, 0)
            for cc in (grp - 3, grp - 2, grp - 1):
                pltpu.make_async_copy(rbufs[cc % 3],
                                      acc.at[didx.at[0, 0, cc]],
                                      ssems[cc % 3]).wait()
            return carry

        lax.fori_loop(0, ngrp, group, 0)
        plsc.subcore_barrier()

        @pl.when(sid < 15)
        def _():
            pltpu.sync_copy(acc.at[pl.ds(sid * rpt8, rpt8)],
                            out_hbm.at[cid, pl.ds(sid * rpt8, rpt8)])

        @pl.when(sid == 15)
        def _():
            pltpu.sync_copy(acc.at[pl.ds(15 * rpt8, last)],
                            out_hbm.at[cid, pl.ds(15 * rpt8, last)])

    return k(x, src.reshape(nw, ngrp, grp, blk), dst.reshape(nw, ngrp, grp, blk),
             w.reshape(nw, ngrp, grp, blk), zeros)


def _mid_call(parts, W1, b1, a1, W2):
    _, n, f = parts.shape
    h = W1.shape[1]
    bm = 400
    gm = n // bm

    def body(p_ref, w1_ref, b1_ref, w2_ref, a1_ref, y_ref):
        agg = p_ref[0] + p_ref[1]
        t = jnp.dot(agg, w1_ref[...], preferred_element_type=jnp.float32)
        t = t + b1_ref[...]
        al = a1_ref[0]
        t = jnp.where(t >= 0, t, al * t)
        y_ref[...] = jnp.dot(t, w2_ref[...], preferred_element_type=jnp.float32)

    return pl.pallas_call(
        body,
        grid=(gm,),
        in_specs=[
            pl.BlockSpec((2, bm, f), lambda i: (0, i, 0)),
            pl.BlockSpec((f, h), lambda i: (0, 0)),
            pl.BlockSpec((1, h), lambda i: (0, 0)),
            pl.BlockSpec((h, f), lambda i: (0, 0)),
            pl.BlockSpec(memory_space=pltpu.SMEM),
        ],
        out_specs=pl.BlockSpec((bm, f), lambda i: (i, 0)),
        out_shape=jax.ShapeDtypeStruct((n, f), jnp.float32),
    )(parts, W1, b1.reshape(1, h), W2, a1.reshape(1))


def _final_call(parts, b2, a2, mskc):
    _, n, f = parts.shape
    bm = 400
    gm = n // bm

    def body(p_ref, b2_ref, m_ref, a2_ref, h_ref, cs_ref, ms_ref):
        i = pl.program_id(0)
        t = p_ref[0] + p_ref[1] + b2_ref[...]
        al = a2_ref[0]
        hh = jnp.where(t >= 0, t, al * t)
        h_ref[...] = hh
        m = m_ref[...]

        @pl.when(i == 0)
        def _():
            cs_ref[...] = jnp.zeros_like(cs_ref)
            ms_ref[...] = jnp.zeros_like(ms_ref)

        cs_ref[...] += jnp.sum(hh * m, axis=0, keepdims=True)
        ms_ref[...] += jnp.sum(m, axis=0, keepdims=True)

    return pl.pallas_call(
        body,
        grid=(gm,),
        in_specs=[
            pl.BlockSpec((2, bm, f), lambda i: (0, i, 0)),
            pl.BlockSpec((1, f), lambda i: (0, 0)),
            pl.BlockSpec((bm, 1), lambda i: (i, 0)),
            pl.BlockSpec(memory_space=pltpu.SMEM),
        ],
        out_specs=[
            pl.BlockSpec((bm, f), lambda i: (i, 0)),
            pl.BlockSpec((1, f), lambda i: (0, 0)),
            pl.BlockSpec((1, 1), lambda i: (0, 0)),
        ],
        out_shape=[
            jax.ShapeDtypeStruct((n, f), jnp.float32),
            jax.ShapeDtypeStruct((1, f), jnp.float32),
            jax.ShapeDtypeStruct((1, 1), jnp.float32),
        ],
    )(parts, b2.reshape(1, f), mskc, a2.reshape(1))


def kernel(seq, edge_index, edge_weight, sparse, msk, W1, b1, a1, W2, b2, a2):
    n, f = seq.shape[1], seq.shape[2]
    x0 = seq[0]
    src = edge_index[1]
    dst = edge_index[0]
    zeros = jnp.zeros((n, f), jnp.float32)
    p1 = _spmm_call(x0, src, dst, edge_weight, zeros)
    y = _mid_call(p1, W1, b1, a1, W2)
    p2 = _spmm_call(y, src, dst, edge_weight, zeros)
    hout, cs, ms = _final_call(p2, b2, a2, msk.reshape(n, 1))
    return (hout[None], cs / ms)


# speedup vs baseline: 15.8161x; 1.0312x over previous
import functools

import jax
import jax.numpy as jnp
from jax import lax
from jax.experimental import pallas as pl
from jax.experimental.pallas import tpu as pltpu
from jax.experimental.pallas import tpu_sc as plsc


def _spmm_call(x, src, dst, w, zeros):
    n, f = x.shape
    e = src.shape[0]
    nw = 32
    epw = e // nw
    blk = 80
    nch = epw // blk
    ngrp = 5
    grp = nch // ngrp
    rpt8 = (n // 16) // 8 * 8
    last = n - 15 * rpt8
    mesh = plsc.VectorSubcoreMesh(core_axis_name="c", subcore_axis_name="s")

    @functools.partial(
        pl.kernel,
        mesh=mesh,
        out_type=jax.ShapeDtypeStruct((2, n, f), jnp.float32),
        scratch_types=[
            pltpu.VMEM((1, 1, grp, blk), jnp.int32),
            pltpu.VMEM((1, 1, grp, blk), jnp.int32),
            pltpu.VMEM((1, 1, grp, blk), jnp.float32),
            pltpu.VMEM((blk, f), jnp.float32),
            pltpu.VMEM((blk, f), jnp.float32),
            pltpu.VMEM((blk, f), jnp.float32),
            pltpu.VMEM_SHARED((n, f), jnp.float32),
            pltpu.SemaphoreType.DMA,
            pltpu.SemaphoreType.DMA,
            pltpu.SemaphoreType.DMA,
            pltpu.SemaphoreType.DMA,
            pltpu.SemaphoreType.DMA,
            pltpu.SemaphoreType.DMA,
            pltpu.SemaphoreType.DMA,
        ],
    )
    def k(x_hbm, src_hbm, dst_hbm, w_hbm, z_hbm, out_hbm,
          sidx, didx, wv, rows0, rows1, rows2, acc,
          gs0, gs1, gs2, ss0, ss1, ss2, ms):
        cid = lax.axis_index("c")
        sid = lax.axis_index("s")
        wid = sid * 2 + cid
        @pl.when(sid < 15)
        def _():
            pltpu.sync_copy(z_hbm.at[pl.ds(sid * rpt8, rpt8)],
                            acc.at[pl.ds(sid * rpt8, rpt8)])

        @pl.when(sid == 15)
        def _():
            pltpu.sync_copy(z_hbm.at[pl.ds(15 * rpt8, last)],
                            acc.at[pl.ds(15 * rpt8, last)])

        plsc.subcore_barrier()

        def scale(rows, ci):
            def sgrp(g, c2):
                wgrp = wv[0, 0, ci, pl.ds(g * 16, 16)]
                for r2 in range(16):
                    wvec = jnp.full((16,), wgrp[r2], jnp.float32)
                    r = g * 16 + r2
                    for j in range(f // 16):
                        sl = pl.ds(j * 16, 16)
                        rows[r, sl] = rows[r, sl] * wvec
                return c2

            lax.fori_loop(0, blk // 16, sgrp, 0)

        def group(g, carry):
            pltpu.async_copy(src_hbm.at[pl.ds(wid, 1), pl.ds(g, 1)], sidx, ms)
            pltpu.async_copy(dst_hbm.at[pl.ds(wid, 1), pl.ds(g, 1)], didx, ms)
            pltpu.async_copy(w_hbm.at[pl.ds(wid, 1), pl.ds(g, 1)], wv, ms)
            pltpu.make_async_copy(
                src_hbm.at[pl.ds(wid, 1), pl.ds(g, 1)], sidx, ms).wait()
            pltpu.make_async_copy(
                dst_hbm.at[pl.ds(wid, 1), pl.ds(g, 1)], didx, ms).wait()
            pltpu.make_async_copy(
                w_hbm.at[pl.ds(wid, 1), pl.ds(g, 1)], wv, ms).wait()
            rbufs = (rows0, rows1, rows2)
            gsems = (gs0, gs1, gs2)
            ssems = (ss0, ss1, ss2)
            pltpu.async_copy(x_hbm.at[sidx.at[0, 0, 0]], rows0, gs0)
            pltpu.async_copy(x_hbm.at[sidx.at[0, 0, 1]], rows1, gs1)

            def stage(ci, b):
                rows, gs, ss = rbufs[b], gsems[b], ssems[b]
                nb = (b + 2) % 3
                pltpu.make_async_copy(x_hbm.at[sidx.at[0, 0, ci]], rows, gs).wait()
                scale(rows, ci)
                pltpu.async_copy(rows, acc.at[didx.at[0, 0, ci]], ss, add=True)

                @pl.when(ci + 2 < grp)
                def _():
                    @pl.when(ci >= 1)
                    def _():
                        cim1 = jnp.maximum(ci - 1, 0)
                        pltpu.make_async_copy(
                            rbufs[nb], acc.at[didx.at[0, 0, cim1]],
                            ssems[nb]).wait()

                    pltpu.async_copy(x_hbm.at[sidx.at[0, 0, ci + 2]],
                                     rbufs[nb], gsems[nb])

            def triple(t, c2):
                for k3 in range(3):
                    ci = 3 * t + k3

                    @pl.when(ci < grp)
                    def _():
                        stage(ci, k3)

                return c2

            lax.fori_loop(0, (grp + 2) // 3, triple, 0)
            for cc in (grp - 3, grp - 2, grp - 1):
                pltpu.make_async_copy(rbufs[cc % 3],
                                      acc.at[didx.at[0, 0, cc]],
                                      ssems[cc % 3]).wait()
            return carry

        lax.fori_loop(0, ngrp, group, 0)
        plsc.subcore_barrier()

        @pl.when(sid < 15)
        def _():
            pltpu.sync_copy(acc.at[pl.ds(sid * rpt8, rpt8)],
                            out_hbm.at[cid, pl.ds(sid * rpt8, rpt8)])

        @pl.when(sid == 15)
        def _():
            pltpu.sync_copy(acc.at[pl.ds(15 * rpt8, last)],
                            out_hbm.at[cid, pl.ds(15 * rpt8, last)])

    return k(x, src.reshape(nw, ngrp, grp, blk), dst.reshape(nw, ngrp, grp, blk),
             w.reshape(nw, ngrp, grp, blk), zeros)


def _mid_call(parts, W1, b1, a1, W2):
    _, n, f = parts.shape
    h = W1.shape[1]
    bm = 400
    gm = n // bm

    def body(p_ref, w1_ref, b1_ref, w2_ref, a1_ref, y_ref):
        agg = (p_ref[0] + p_ref[1]).astype(jnp.bfloat16)
        t = jnp.dot(agg, w1_ref[...].astype(jnp.bfloat16),
                    preferred_element_type=jnp.float32)
        t = t + b1_ref[...]
        al = a1_ref[0]
        t = jnp.where(t >= 0, t, al * t)
        y_ref[...] = jnp.dot(t.astype(jnp.bfloat16),
                             w2_ref[...].astype(jnp.bfloat16),
                             preferred_element_type=jnp.float32)

    return pl.pallas_call(
        body,
        grid=(gm,),
        in_specs=[
            pl.BlockSpec((2, bm, f), lambda i: (0, i, 0)),
            pl.BlockSpec((f, h), lambda i: (0, 0)),
            pl.BlockSpec((1, h), lambda i: (0, 0)),
            pl.BlockSpec((h, f), lambda i: (0, 0)),
            pl.BlockSpec(memory_space=pltpu.SMEM),
        ],
        out_specs=pl.BlockSpec((bm, f), lambda i: (i, 0)),
        out_shape=jax.ShapeDtypeStruct((n, f), jnp.float32),
    )(parts, W1, b1.reshape(1, h), W2, a1.reshape(1))


def _final_call(parts, b2, a2, mskc):
    _, n, f = parts.shape
    bm = 400
    gm = n // bm

    def body(p_ref, b2_ref, m_ref, a2_ref, h_ref, cs_ref, ms_ref):
        i = pl.program_id(0)
        t = p_ref[0] + p_ref[1] + b2_ref[...]
        al = a2_ref[0]
        hh = jnp.where(t >= 0, t, al * t)
        h_ref[...] = hh
        m = m_ref[...]

        @pl.when(i == 0)
        def _():
            cs_ref[...] = jnp.zeros_like(cs_ref)
            ms_ref[...] = jnp.zeros_like(ms_ref)

        cs_ref[...] += jnp.sum(hh * m, axis=0, keepdims=True)
        ms_ref[...] += jnp.sum(m, axis=0, keepdims=True)

    return pl.pallas_call(
        body,
        grid=(gm,),
        in_specs=[
            pl.BlockSpec((2, bm, f), lambda i: (0, i, 0)),
            pl.BlockSpec((1, f), lambda i: (0, 0)),
            pl.BlockSpec((bm, 1), lambda i: (i, 0)),
            pl.BlockSpec(memory_space=pltpu.SMEM),
        ],
        out_specs=[
            pl.BlockSpec((bm, f), lambda i: (i, 0)),
            pl.BlockSpec((1, f), lambda i: (0, 0)),
            pl.BlockSpec((1, 1), lambda i: (0, 0)),
        ],
        out_shape=[
            jax.ShapeDtypeStruct((n, f), jnp.float32),
            jax.ShapeDtypeStruct((1, f), jnp.float32),
            jax.ShapeDtypeStruct((1, 1), jnp.float32),
        ],
    )(parts, b2.reshape(1, f), mskc, a2.reshape(1))


def kernel(seq, edge_index, edge_weight, sparse, msk, W1, b1, a1, W2, b2, a2):
    n, f = seq.shape[1], seq.shape[2]
    x0 = seq[0]
    src = edge_index[1]
    dst = edge_index[0]
    zeros = jnp.zeros((n, f), jnp.float32)
    p1 = _spmm_call(x0, src, dst, edge_weight, zeros)
    y = _mid_call(p1, W1, b1, a1, W2)
    p2 = _spmm_call(y, src, dst, edge_weight, zeros)
    hout, cs, ms = _final_call(p2, b2, a2, msk.reshape(n, 1))
    return (hout[None], cs / ms)


# speedup vs baseline: 16.4052x; 1.0372x over previous
import functools

import jax
import jax.numpy as jnp
from jax import lax
from jax.experimental import pallas as pl
from jax.experimental.pallas import tpu as pltpu
from jax.experimental.pallas import tpu_sc as plsc


def _spmm_call(x, src, dst, w):
    n, f = x.shape
    e = src.shape[0]
    nw = 32
    epw = e // nw
    blk = 80
    nch = epw // blk
    ngrp = 5
    grp = nch // ngrp
    rpt8 = (n // 16) // 8 * 8
    last = n - 15 * rpt8
    mesh = plsc.VectorSubcoreMesh(core_axis_name="c", subcore_axis_name="s")

    @functools.partial(
        pl.kernel,
        mesh=mesh,
        out_type=jax.ShapeDtypeStruct((2, n, f), jnp.float32),
        scratch_types=[
            pltpu.VMEM((1, 1, grp, blk), jnp.int32),
            pltpu.VMEM((1, 1, grp, blk), jnp.int32),
            pltpu.VMEM((1, 1, grp, blk), jnp.float32),
            pltpu.VMEM((blk, f), jnp.float32),
            pltpu.VMEM((blk, f), jnp.float32),
            pltpu.VMEM((blk, f), jnp.float32),
            pltpu.VMEM_SHARED((n, f), jnp.float32),
            pltpu.SemaphoreType.DMA,
            pltpu.SemaphoreType.DMA,
            pltpu.SemaphoreType.DMA,
            pltpu.SemaphoreType.DMA,
            pltpu.SemaphoreType.DMA,
            pltpu.SemaphoreType.DMA,
            pltpu.SemaphoreType.DMA,
        ],
    )
    def k(x_hbm, src_hbm, dst_hbm, w_hbm, out_hbm,
          sidx, didx, wv, rows0, rows1, rows2, acc,
          gs0, gs1, gs2, ss0, ss1, ss2, ms):
        cid = lax.axis_index("c")
        sid = lax.axis_index("s")
        wid = sid * 2 + cid
        def zrow(r, c2):
            for j in range(f // 16):
                rows0[r, pl.ds(j * 16, 16)] = jnp.zeros((16,), jnp.float32)
            return c2

        lax.fori_loop(0, blk, zrow, 0)

        @pl.when(sid < 15)
        def _():
            for kk in range(7):
                pltpu.async_copy(rows0, acc.at[pl.ds(sid * rpt8 + kk * blk, blk)], ms)
            pltpu.async_copy(rows0.at[pl.ds(0, 64)],
                             acc.at[pl.ds(sid * rpt8 + 7 * blk, 64)], ms)
            for kk in range(7):
                pltpu.make_async_copy(
                    rows0, acc.at[pl.ds(sid * rpt8 + kk * blk, blk)], ms).wait()
            pltpu.make_async_copy(
                rows0.at[pl.ds(0, 64)],
                acc.at[pl.ds(sid * rpt8 + 7 * blk, 64)], ms).wait()

        @pl.when(sid == 15)
        def _():
            for kk in range(8):
                pltpu.async_copy(rows0, acc.at[pl.ds(15 * rpt8 + kk * blk, blk)], ms)
            for kk in range(8):
                pltpu.make_async_copy(
                    rows0, acc.at[pl.ds(15 * rpt8 + kk * blk, blk)], ms).wait()

        plsc.subcore_barrier()

        def scale(rows, ci):
            def sgrp(g, c2):
                wgrp = wv[0, 0, ci, pl.ds(g * 16, 16)]
                for r2 in range(16):
                    wvec = jnp.full((16,), wgrp[r2], jnp.float32)
                    r = g * 16 + r2
                    for j in range(f // 16):
                        sl = pl.ds(j * 16, 16)
                        rows[r, sl] = rows[r, sl] * wvec
                return c2

            lax.fori_loop(0, blk // 16, sgrp, 0)

        def group(g, carry):
            pltpu.async_copy(src_hbm.at[pl.ds(wid, 1), pl.ds(g, 1)], sidx, ms)
            pltpu.async_copy(dst_hbm.at[pl.ds(wid, 1), pl.ds(g, 1)], didx, ms)
            pltpu.async_copy(w_hbm.at[pl.ds(wid, 1), pl.ds(g, 1)], wv, ms)
            pltpu.make_async_copy(
                src_hbm.at[pl.ds(wid, 1), pl.ds(g, 1)], sidx, ms).wait()
            pltpu.make_async_copy(
                dst_hbm.at[pl.ds(wid, 1), pl.ds(g, 1)], didx, ms).wait()
            pltpu.make_async_copy(
                w_hbm.at[pl.ds(wid, 1), pl.ds(g, 1)], wv, ms).wait()
            rbufs = (rows0, rows1, rows2)
            gsems = (gs0, gs1, gs2)
            ssems = (ss0, ss1, ss2)
            pltpu.async_copy(x_hbm.at[sidx.at[0, 0, 0]], rows0, gs0)
            pltpu.async_copy(x_hbm.at[sidx.at[0, 0, 1]], rows1, gs1)

            def stage(ci, b):
                rows, gs, ss = rbufs[b], gsems[b], ssems[b]
                nb = (b + 2) % 3
                pltpu.make_async_copy(x_hbm.at[sidx.at[0, 0, ci]], rows, gs).wait()
                scale(rows, ci)
                pltpu.async_copy(rows, acc.at[didx.at[0, 0, ci]], ss, add=True)

                @pl.when(ci + 2 < grp)
                def _():
                    @pl.when(ci >= 1)
                    def _():
                        cim1 = jnp.maximum(ci - 1, 0)
                        pltpu.make_async_copy(
                            rbufs[nb], acc.at[didx.at[0, 0, cim1]],
                            ssems[nb]).wait()

                    pltpu.async_copy(x_hbm.at[sidx.at[0, 0, ci + 2]],
                                     rbufs[nb], gsems[nb])

            def triple(t, c2):
                for k3 in range(3):
                    ci = 3 * t + k3

                    @pl.when(ci < grp)
                    def _():
                        stage(ci, k3)

                return c2

            lax.fori_loop(0, (grp + 2) // 3, triple, 0)
            for cc in (grp - 3, grp - 2, grp - 1):
                pltpu.make_async_copy(rbufs[cc % 3],
                                      acc.at[didx.at[0, 0, cc]],
                                      ssems[cc % 3]).wait()
            return carry

        lax.fori_loop(0, ngrp, group, 0)
        plsc.subcore_barrier()

        @pl.when(sid < 15)
        def _():
            pltpu.sync_copy(acc.at[pl.ds(sid * rpt8, rpt8)],
                            out_hbm.at[cid, pl.ds(sid * rpt8, rpt8)])

        @pl.when(sid == 15)
        def _():
            pltpu.sync_copy(acc.at[pl.ds(15 * rpt8, last)],
                            out_hbm.at[cid, pl.ds(15 * rpt8, last)])

    return k(x, src.reshape(nw, ngrp, grp, blk), dst.reshape(nw, ngrp, grp, blk),
             w.reshape(nw, ngrp, grp, blk))


def _mid_call(parts, W1, b1, a1, W2):
    _, n, f = parts.shape
    h = W1.shape[1]
    bm = 400
    gm = n // bm

    def body(p_ref, w1_ref, b1_ref, w2_ref, a1_ref, y_ref):
        agg = (p_ref[0] + p_ref[1]).astype(jnp.bfloat16)
        t = jnp.dot(agg, w1_ref[...].astype(jnp.bfloat16),
                    preferred_element_type=jnp.float32)
        t = t + b1_ref[...]
        al = a1_ref[0]
        t = jnp.where(t >= 0, t, al * t)
        y_ref[...] = jnp.dot(t.astype(jnp.bfloat16),
                             w2_ref[...].astype(jnp.bfloat16),
                             preferred_element_type=jnp.float32)

    return pl.pallas_call(
        body,
        grid=(gm,),
        in_specs=[
            pl.BlockSpec((2, bm, f), lambda i: (0, i, 0)),
            pl.BlockSpec((f, h), lambda i: (0, 0)),
            pl.BlockSpec((1, h), lambda i: (0, 0)),
            pl.BlockSpec((h, f), lambda i: (0, 0)),
            pl.BlockSpec(memory_space=pltpu.SMEM),
        ],
        out_specs=pl.BlockSpec((bm, f), lambda i: (i, 0)),
        out_shape=jax.ShapeDtypeStruct((n, f), jnp.float32),
    )(parts, W1, b1.reshape(1, h), W2, a1.reshape(1))


def _final_call(parts, b2, a2, mskc):
    _, n, f = parts.shape
    bm = 400
    gm = n // bm

    def body(p_ref, b2_ref, m_ref, a2_ref, h_ref, cs_ref, csum, mssum):
        i = pl.program_id(0)
        t = p_ref[0] + p_ref[1] + b2_ref[...]
        al = a2_ref[0]
        hh = jnp.where(t >= 0, t, al * t)
        h_ref[...] = hh
        m = m_ref[...]

        @pl.when(i == 0)
        def _():
            csum[...] = jnp.zeros_like(csum)
            mssum[...] = jnp.zeros_like(mssum)

        csum[...] += jnp.sum(hh * m, axis=0, keepdims=True)
        mssum[...] += jnp.sum(m, axis=0, keepdims=True)

        @pl.when(i == pl.num_programs(0) - 1)
        def _():
            cs_ref[...] = csum[...] / mssum[0, 0]

    return pl.pallas_call(
        body,
        grid=(gm,),
        in_specs=[
            pl.BlockSpec((2, bm, f), lambda i: (0, i, 0)),
            pl.BlockSpec((1, f), lambda i: (0, 0)),
            pl.BlockSpec((bm, 1), lambda i: (i, 0)),
            pl.BlockSpec(memory_space=pltpu.SMEM),
        ],
        out_specs=[
            pl.BlockSpec((bm, f), lambda i: (i, 0)),
            pl.BlockSpec((1, f), lambda i: (0, 0)),
        ],
        out_shape=[
            jax.ShapeDtypeStruct((n, f), jnp.float32),
            jax.ShapeDtypeStruct((1, f), jnp.float32),
        ],
        scratch_shapes=[
            pltpu.VMEM((1, f), jnp.float32),
            pltpu.VMEM((1, 1), jnp.float32),
        ],
    )(parts, b2.reshape(1, f), mskc, a2.reshape(1))


def kernel(seq, edge_index, edge_weight, sparse, msk, W1, b1, a1, W2, b2, a2):
    n, f = seq.shape[1], seq.shape[2]
    x0 = seq[0]
    src = edge_index[1]
    dst = edge_index[0]
    p1 = _spmm_call(x0, src, dst, edge_weight)
    y = _mid_call(p1, W1, b1, a1, W2)
    p2 = _spmm_call(y, src, dst, edge_weight)
    hout, cs = _final_call(p2, b2, a2, msk.reshape(n, 1))
    return (hout[None], cs)
